# Initial kernel scaffold; baseline (speedup 1.0000x reference)
import jax
import jax.numpy as jnp
from jax.experimental import pallas as pl


def kernel(H, distance_matrix, Wq, Wk, Wv, Wo, bo):
    raise NotImplementedError("write your pallas kernel here")


# speedup vs baseline: 7.9515x; 7.9515x over previous
import functools

import jax
import jax.numpy as jnp
from jax.experimental import pallas as pl
from jax.experimental.pallas import tpu as pltpu

N = 4096
C = 512
HD = 512
K_NEIGH = 16
QBLK = 256
SCALE = HD ** (-0.5)


def _body(h_ref, d_ref, wq_ref, wk_ref, wv_ref, wo_ref, bo_ref, o_ref,
          k_scr, v_scr):
    i = pl.program_id(0)

    @pl.when(i == 0)
    def _():
        h_all = h_ref[...]
        k_scr[...] = jax.lax.dot(h_all, wk_ref[...],
                                 preferred_element_type=jnp.float32)
        v_scr[...] = jax.lax.dot(h_all, wv_ref[...],
                                 preferred_element_type=jnp.float32)

    hb = h_ref[pl.ds(i * QBLK, QBLK), :]
    q = jax.lax.dot(hb, wq_ref[...], preferred_element_type=jnp.float32)

    d = d_ref[...]
    iota = jax.lax.broadcasted_iota(jnp.int32, (QBLK, N), 1)
    mask = jnp.zeros((QBLK, N), dtype=jnp.bool_)
    for _ in range(K_NEIGH):
        m = jnp.min(d, axis=1, keepdims=True)
        cand = jnp.where(d == m, iota, N)
        sel_idx = jnp.min(cand, axis=1, keepdims=True)
        sel = iota == sel_idx
        mask = jnp.logical_or(mask, sel)
        d = jnp.where(sel, jnp.inf, d)

    s = jax.lax.dot_general(q, k_scr[...], (((1,), (1,)), ((), ())),
                            preferred_element_type=jnp.float32) * SCALE
    logits = jnp.where(mask, s, -jnp.inf)
    mx = jnp.max(logits, axis=1, keepdims=True)
    p = jnp.exp(logits - mx)
    attn = p / jnp.sum(p, axis=1, keepdims=True)

    he = jax.lax.dot_general(attn, v_scr[...], (((1,), (0,)), ((), ())),
                             preferred_element_type=jnp.float32)
    o_ref[...] = (jax.lax.dot(he, wo_ref[...],
                              preferred_element_type=jnp.float32)
                  + bo_ref[...] + hb)


@jax.jit
def kernel(H, distance_matrix, Wq, Wk, Wv, Wo, bo):
    grid = (N // QBLK,)
    out = pl.pallas_call(
        _body,
        grid=grid,
        in_specs=[
            pl.BlockSpec((N, C), lambda i: (0, 0)),
            pl.BlockSpec((QBLK, N), lambda i: (i, 0)),
            pl.BlockSpec((C, HD), lambda i: (0, 0)),
            pl.BlockSpec((C, HD), lambda i: (0, 0)),
            pl.BlockSpec((C, C), lambda i: (0, 0)),
            pl.BlockSpec((C, C), lambda i: (0, 0)),
            pl.BlockSpec((1, C), lambda i: (0, 0)),
        ],
        out_specs=pl.BlockSpec((QBLK, C), lambda i: (i, 0)),
        out_shape=jax.ShapeDtypeStruct((N, C), jnp.float32),
        scratch_shapes=[
            pltpu.VMEM((N, HD), jnp.float32),
            pltpu.VMEM((N, C), jnp.float32),
        ],
    )(H, distance_matrix, Wq, Wk, Wv, Wo, bo.reshape(1, C))
    return out


# speedup vs baseline: 20.3429x; 2.5584x over previous
import functools

import jax
import jax.numpy as jnp
from jax.experimental import pallas as pl
from jax.experimental.pallas import tpu as pltpu

N = 4096
C = 512
HD = 512
K_NEIGH = 16
QBLK = 256
SCALE = HD ** (-0.5)


def _body(h_ref, d_ref, wq_ref, wk_ref, wv_ref, wo_ref, bo_ref, o_ref,
          k_scr, v_scr):
    i = pl.program_id(0)

    @pl.when(i == 0)
    def _():
        h_all = h_ref[...]
        k_scr[...] = jax.lax.dot(h_all, wk_ref[...],
                                 preferred_element_type=jnp.float32)
        v_scr[...] = jax.lax.dot(h_all, wv_ref[...],
                                 preferred_element_type=jnp.float32)

    hb = h_ref[pl.ds(i * QBLK, QBLK), :]
    q = jax.lax.dot(hb, wq_ref[...], preferred_element_type=jnp.float32)

    d = d_ref[...]
    for _ in range(K_NEIGH):
        m = jnp.min(d, axis=1, keepdims=True)
        d = jnp.where(d == m, jnp.inf, d)
    mask = d == jnp.inf

    s = jax.lax.dot_general(q, k_scr[...], (((1,), (1,)), ((), ())),
                            preferred_element_type=jnp.float32) * SCALE
    logits = jnp.where(mask, s, -jnp.inf)
    mx = jnp.max(logits, axis=1, keepdims=True)
    p = jnp.exp(logits - mx)
    attn = p / jnp.sum(p, axis=1, keepdims=True)

    he = jax.lax.dot_general(attn, v_scr[...], (((1,), (0,)), ((), ())),
                             preferred_element_type=jnp.float32)
    o_ref[...] = (jax.lax.dot(he, wo_ref[...],
                              preferred_element_type=jnp.float32)
                  + bo_ref[...] + hb)


@jax.jit
def kernel(H, distance_matrix, Wq, Wk, Wv, Wo, bo):
    grid = (N // QBLK,)
    out = pl.pallas_call(
        _body,
        grid=grid,
        in_specs=[
            pl.BlockSpec((N, C), lambda i: (0, 0)),
            pl.BlockSpec((QBLK, N), lambda i: (i, 0)),
            pl.BlockSpec((C, HD), lambda i: (0, 0)),
            pl.BlockSpec((C, HD), lambda i: (0, 0)),
            pl.BlockSpec((C, C), lambda i: (0, 0)),
            pl.BlockSpec((C, C), lambda i: (0, 0)),
            pl.BlockSpec((1, C), lambda i: (0, 0)),
        ],
        out_specs=pl.BlockSpec((QBLK, C), lambda i: (i, 0)),
        out_shape=jax.ShapeDtypeStruct((N, C), jnp.float32),
        scratch_shapes=[
            pltpu.VMEM((N, HD), jnp.float32),
            pltpu.VMEM((N, C), jnp.float32),
        ],
    )(H, distance_matrix, Wq, Wk, Wv, Wo, bo.reshape(1, C))
    return out


# speedup vs baseline: 20.4864x; 1.0071x over previous
import functools

import jax
import jax.numpy as jnp
from jax.experimental import pallas as pl
from jax.experimental.pallas import tpu as pltpu

N = 4096
C = 512
HD = 512
K_NEIGH = 16
QBLK = 256
SCALE = HD ** (-0.5)


def _body(h_ref, d_ref, wq_ref, wk_ref, wv_ref, wo_ref, bo_ref, o_ref,
          k_scr, v_scr):
    i = pl.program_id(0)

    @pl.when(i == 0)
    def _():
        h_all = h_ref[...].astype(jnp.bfloat16)
        k_scr[...] = jax.lax.dot(h_all, wk_ref[...].astype(jnp.bfloat16),
                                 preferred_element_type=jnp.float32
                                 ).astype(jnp.bfloat16)
        v_scr[...] = jax.lax.dot(h_all, wv_ref[...].astype(jnp.bfloat16),
                                 preferred_element_type=jnp.float32
                                 ).astype(jnp.bfloat16)

    hb = h_ref[pl.ds(i * QBLK, QBLK), :]
    q = jax.lax.dot(hb.astype(jnp.bfloat16),
                    wq_ref[...].astype(jnp.bfloat16),
                    preferred_element_type=jnp.float32).astype(jnp.bfloat16)

    d = d_ref[...]
    for _ in range(K_NEIGH):
        m = jnp.min(d, axis=1, keepdims=True)
        d = jnp.where(d == m, jnp.inf, d)
    mask = d == jnp.inf

    s = jax.lax.dot_general(q, k_scr[...], (((1,), (1,)), ((), ())),
                            preferred_element_type=jnp.float32) * SCALE
    logits = jnp.where(mask, s, -jnp.inf)
    mx = jnp.max(logits, axis=1, keepdims=True)
    p = jnp.exp(logits - mx)
    attn = p / jnp.sum(p, axis=1, keepdims=True)

    he = jax.lax.dot_general(attn.astype(jnp.bfloat16), v_scr[...],
                             (((1,), (0,)), ((), ())),
                             preferred_element_type=jnp.float32
                             ).astype(jnp.bfloat16)
    o_ref[...] = (jax.lax.dot(he, wo_ref[...].astype(jnp.bfloat16),
                              preferred_element_type=jnp.float32)
                  + bo_ref[...] + hb)


@jax.jit
def kernel(H, distance_matrix, Wq, Wk, Wv, Wo, bo):
    grid = (N // QBLK,)
    out = pl.pallas_call(
        _body,
        grid=grid,
        in_specs=[
            pl.BlockSpec((N, C), lambda i: (0, 0)),
            pl.BlockSpec((QBLK, N), lambda i: (i, 0)),
            pl.BlockSpec((C, HD), lambda i: (0, 0)),
            pl.BlockSpec((C, HD), lambda i: (0, 0)),
            pl.BlockSpec((C, C), lambda i: (0, 0)),
            pl.BlockSpec((C, C), lambda i: (0, 0)),
            pl.BlockSpec((1, C), lambda i: (0, 0)),
        ],
        out_specs=pl.BlockSpec((QBLK, C), lambda i: (i, 0)),
        out_shape=jax.ShapeDtypeStruct((N, C), jnp.float32),
        scratch_shapes=[
            pltpu.VMEM((N, HD), jnp.bfloat16),
            pltpu.VMEM((N, C), jnp.bfloat16),
        ],
    )(H, distance_matrix, Wq, Wk, Wv, Wo, bo.reshape(1, C))
    return out


# speedup vs baseline: 20.8812x; 1.0193x over previous
import functools

import jax
import jax.numpy as jnp
from jax.experimental import pallas as pl
from jax.experimental.pallas import tpu as pltpu

N = 4096
C = 512
HD = 512
K_NEIGH = 16
QBLK = 256
SCALE = HD ** (-0.5)


def _body(h_ref, d_ref, wq_ref, wk_ref, wv_ref, wo_ref, bo_ref, o_ref,
          k_scr, v_scr):
    i = pl.program_id(0)

    @pl.when(i == 0)
    def _():
        h_all = h_ref[...].astype(jnp.bfloat16)
        k_scr[...] = jax.lax.dot(h_all, wk_ref[...].astype(jnp.bfloat16),
                                 preferred_element_type=jnp.float32
                                 ).astype(jnp.bfloat16)
        v_scr[...] = jax.lax.dot(h_all, wv_ref[...].astype(jnp.bfloat16),
                                 preferred_element_type=jnp.float32
                                 ).astype(jnp.bfloat16)

    hb = h_ref[pl.ds(i * QBLK, QBLK), :]
    q = jax.lax.dot(hb.astype(jnp.bfloat16),
                    wq_ref[...].astype(jnp.bfloat16),
                    preferred_element_type=jnp.float32).astype(jnp.bfloat16)

    d = d_ref[...]
    t = jnp.full((QBLK, 1), -jnp.inf, dtype=jnp.float32)
    for _ in range(K_NEIGH):
        t = jnp.min(jnp.where(d > t, d, jnp.inf), axis=1, keepdims=True)
    mask = d <= t

    s = jax.lax.dot_general(q, k_scr[...], (((1,), (1,)), ((), ())),
                            preferred_element_type=jnp.float32) * SCALE
    logits = jnp.where(mask, s, -jnp.inf)
    mx = jnp.max(logits, axis=1, keepdims=True)
    p = jnp.exp(logits - mx)
    attn = p / jnp.sum(p, axis=1, keepdims=True)

    he = jax.lax.dot_general(attn.astype(jnp.bfloat16), v_scr[...],
                             (((1,), (0,)), ((), ())),
                             preferred_element_type=jnp.float32
                             ).astype(jnp.bfloat16)
    o_ref[...] = (jax.lax.dot(he, wo_ref[...].astype(jnp.bfloat16),
                              preferred_element_type=jnp.float32)
                  + bo_ref[...] + hb)


@jax.jit
def kernel(H, distance_matrix, Wq, Wk, Wv, Wo, bo):
    grid = (N // QBLK,)
    out = pl.pallas_call(
        _body,
        grid=grid,
        in_specs=[
            pl.BlockSpec((N, C), lambda i: (0, 0)),
            pl.BlockSpec((QBLK, N), lambda i: (i, 0)),
            pl.BlockSpec((C, HD), lambda i: (0, 0)),
            pl.BlockSpec((C, HD), lambda i: (0, 0)),
            pl.BlockSpec((C, C), lambda i: (0, 0)),
            pl.BlockSpec((C, C), lambda i: (0, 0)),
            pl.BlockSpec((1, C), lambda i: (0, 0)),
        ],
        out_specs=pl.BlockSpec((QBLK, C), lambda i: (i, 0)),
        out_shape=jax.ShapeDtypeStruct((N, C), jnp.float32),
        scratch_shapes=[
            pltpu.VMEM((N, HD), jnp.bfloat16),
            pltpu.VMEM((N, C), jnp.bfloat16),
        ],
    )(H, distance_matrix, Wq, Wk, Wv, Wo, bo.reshape(1, C))
    return out
